# Initial kernel scaffold; baseline (speedup 1.0000x reference)
import jax
import jax.numpy as jnp
from jax.experimental import pallas as pl


def kernel(table, input_labels, pos_labels, neg_labels, syn_word_idxs, ant_word_idxs, valid_syn, valid_ant, upsilon, eta0, eta):
    raise NotImplementedError("write your pallas kernel here")


# speedup vs baseline: 1.9220x; 1.9220x over previous
import functools

import jax
import jax.numpy as jnp
from jax import lax
from jax.experimental import pallas as pl
from jax.experimental.pallas import tpu as pltpu
from jax.experimental.pallas import tpu_sc as plsc

_D = 64
_B = 4096
_NW = 32
_PER_W = _B // _NW
_E = 16
_G = _PER_W // _E
_R = 40
_CROWS = _E * _R
_OUTC = 48

_mesh = plsc.VectorSubcoreMesh(core_axis_name="c", subcore_axis_name="s")


@functools.partial(
    pl.kernel,
    mesh=_mesh,
    out_type=jax.ShapeDtypeStruct((_B, _OUTC), jnp.float32),
    scratch_types=[
        pltpu.VMEM((_E,), jnp.int32),
        pltpu.VMEM((128,), jnp.int32),
        pltpu.VMEM((_E, _D), jnp.float32),
        pltpu.VMEM((_CROWS + 16, _D), jnp.float32),
        pltpu.VMEM((_E, _OUTC), jnp.float32),
        pltpu.SemaphoreType.DMA,
    ],
    compiler_params=pltpu.CompilerParams(
        needs_layout_passes=False, use_tc_tiling_on_sc=False),
)
def _sc_dots(table, in_idx, cmb_idx, out, iidx, cidx, in_v, comb_v, out_v, sem):
    wid = lax.axis_index("s") * 2 + lax.axis_index("c")
    iota = lax.iota(jnp.int32, 16)
    lane_c = [iota + (16 * j) for j in range(3)]

    def group(g, carry):
        base = wid * _PER_W + g * _E
        pltpu.sync_copy(in_idx.at[pl.ds(base, _E)], iidx)
        pltpu.async_copy(table.at[iidx], in_v, sem).wait()
        cb = base * _R
        for k in range(_CROWS // 128):
            pltpu.sync_copy(cmb_idx.at[pl.ds(cb + 128 * k, 128)], cidx)
            pltpu.async_copy(table.at[cidx], comb_v.at[pl.ds(128 * k, 128)], sem).wait()

        def elem(e, c2):
            splat_e = jnp.full((16,), e, jnp.int32)
            rows = [splat_e * _R + lane_c[j] for j in range(3)]
            dvec = jnp.zeros((16,), jnp.int32)
            acc = [jnp.zeros((16,), jnp.float32) for _ in range(3)]
            for _ in range(_D):
                b = plsc.load_gather(in_v, [splat_e, dvec])
                for j in range(3):
                    v = plsc.load_gather(comb_v, [rows[j], dvec])
                    acc[j] = acc[j] + v * b
                dvec = dvec + 1
            for j in range(3):
                out_v[e, pl.ds(16 * j, 16)] = acc[j]
            return c2

        lax.fori_loop(0, _E, elem, 0)
        pltpu.sync_copy(out_v, out.at[pl.ds(base, _E)])
        return carry

    lax.fori_loop(0, _G, group, 0)


def _tc_body(dots_ref, vs_ref, va_ref, out_ref):
    x = dots_ref[...]
    s1 = (jnp.sum(jax.nn.log_sigmoid(x[:, 0:10] ** 2))
          + jnp.sum(jax.nn.log_sigmoid(-(x[:, 10:30] ** 2))))
    syn = jnp.sum(x[:, 30:35], axis=1, keepdims=True) * vs_ref[...]
    ant = jnp.sum(x[:, 35:40], axis=1, keepdims=True) * va_ref[...]
    s2 = jnp.sum(ant - syn)
    out_ref[...] = jnp.concatenate(
        [jnp.reshape(s1, (1, 1)), jnp.reshape(s2, (1, 1))], axis=1)


_tc_reduce = pl.pallas_call(
    _tc_body,
    out_shape=jax.ShapeDtypeStruct((1, 2), jnp.float32),
)


def kernel(table, input_labels, pos_labels, neg_labels, syn_word_idxs,
           ant_word_idxs, valid_syn, valid_ant, upsilon, eta0, eta):
    table = table.astype(jnp.float32)
    ii = input_labels.astype(jnp.int32)
    cmb = jnp.concatenate(
        [pos_labels.astype(jnp.int32), neg_labels.astype(jnp.int32),
         syn_word_idxs.astype(jnp.int32), ant_word_idxs.astype(jnp.int32)],
        axis=1).reshape(-1)
    dots = _sc_dots(table, ii, cmb)
    s = _tc_reduce(dots,
                   valid_syn.astype(jnp.float32).reshape(_B, 1),
                   valid_ant.astype(jnp.float32).reshape(_B, 1))
    bf = jnp.float32(_B)
    loss = eta0 * (s[0, 0] / bf) - eta * jnp.maximum(
        jnp.float32(0.0), upsilon + s[0, 1] / bf)
    return -loss


# speedup vs baseline: 2.2777x; 1.1851x over previous
import functools

import jax
import jax.numpy as jnp
from jax import lax
from jax.experimental import pallas as pl
from jax.experimental.pallas import tpu as pltpu
from jax.experimental.pallas import tpu_sc as plsc

_D = 64
_B = 4096
_NW = 32
_PER_W = _B // _NW
_E = 16
_G = _PER_W // _E
_CROWS = 640
_OUTC = 48

_mesh = plsc.VectorSubcoreMesh(core_axis_name="c", subcore_axis_name="s")


@functools.partial(
    pl.kernel,
    mesh=_mesh,
    out_type=jax.ShapeDtypeStruct((_B, _OUTC), jnp.float32),
    scratch_types=[
        pltpu.VMEM((_PER_W,), jnp.int32),
        pltpu.VMEM((_PER_W * 10,), jnp.int32),
        pltpu.VMEM((_PER_W * 20,), jnp.int32),
        pltpu.VMEM((_PER_W * 5,), jnp.int32),
        pltpu.VMEM((_PER_W * 5,), jnp.int32),
        pltpu.VMEM((_E, _D), jnp.float32),
        pltpu.VMEM((_E, _D), jnp.float32),
        pltpu.VMEM((_CROWS, _D), jnp.float32),
        pltpu.VMEM((_CROWS, _D), jnp.float32),
        pltpu.VMEM((_E, _OUTC), jnp.float32),
        pltpu.VMEM((_E, _OUTC), jnp.float32),
        pltpu.SemaphoreType.DMA,
        pltpu.SemaphoreType.DMA,
        pltpu.SemaphoreType.DMA,
    ],
    compiler_params=pltpu.CompilerParams(
        needs_layout_passes=False, use_tc_tiling_on_sc=False),
)
def _sc_dots(in_idx, pos_idx, neg_idx, syn_idx, ant_idx, table, out,
             iidx, pidx, nidx, sidx, aidx,
             in_v0, in_v1, cv0, cv1, ov0, ov1, sem0, sem1, osem):
    wid = lax.axis_index("s") * 2 + lax.axis_index("c")
    iota = lax.iota(jnp.int32, 16)

    mult = [
        jnp.where(iota < 10, 10, 20),
        jnp.where(iota < 14, 20, 5),
        jnp.where(iota < 8, 5, 0),
    ]
    base = [
        jnp.where(iota < 10, iota, 150 + iota),
        jnp.where(iota < 14, 166 + iota, 466 + iota),
        jnp.where(iota < 3, 482 + iota, jnp.where(iota < 8, 557 + iota, 0)),
    ]

    wbase = wid * _PER_W
    pltpu.sync_copy(in_idx.at[pl.ds(wbase, _PER_W)], iidx)
    pltpu.sync_copy(pos_idx.at[pl.ds(wbase * 10, _PER_W * 10)], pidx)
    pltpu.sync_copy(neg_idx.at[pl.ds(wbase * 20, _PER_W * 20)], nidx)
    pltpu.sync_copy(syn_idx.at[pl.ds(wbase * 5, _PER_W * 5)], sidx)
    pltpu.sync_copy(ant_idx.at[pl.ds(wbase * 5, _PER_W * 5)], aidx)

    in_bufs = (in_v0, in_v1)
    c_bufs = (cv0, cv1)
    o_bufs = (ov0, ov1)
    sems = (sem0, sem1)

    def fire(g):
        cb, sem = c_bufs[g % 2], sems[g % 2]
        cps = [
            pltpu.async_copy(
                table.at[iidx.at[pl.ds(g * 16, 16)]], in_bufs[g % 2], sem),
            pltpu.async_copy(
                table.at[pidx.at[pl.ds(g * 160, 128)]],
                cb.at[pl.ds(0, 128)], sem),
            pltpu.async_copy(
                table.at[pidx.at[pl.ds(g * 160 + 128, 32)]],
                cb.at[pl.ds(128, 32)], sem),
            pltpu.async_copy(
                table.at[nidx.at[pl.ds(g * 320, 128)]],
                cb.at[pl.ds(160, 128)], sem),
            pltpu.async_copy(
                table.at[nidx.at[pl.ds(g * 320 + 128, 128)]],
                cb.at[pl.ds(288, 128)], sem),
            pltpu.async_copy(
                table.at[nidx.at[pl.ds(g * 320 + 256, 64)]],
                cb.at[pl.ds(416, 64)], sem),
            pltpu.async_copy(
                table.at[sidx.at[pl.ds(g * 80, 80)]],
                cb.at[pl.ds(480, 80)], sem),
            pltpu.async_copy(
                table.at[aidx.at[pl.ds(g * 80, 80)]],
                cb.at[pl.ds(560, 80)], sem),
        ]
        return cps

    def compute(g):
        in_v, comb_v, out_v = in_bufs[g % 2], c_bufs[g % 2], o_bufs[g % 2]

        def elem(e, c2):
            splat_e = jnp.full((16,), e, jnp.int32)
            rows = [splat_e * mult[j] + base[j] for j in range(3)]
            dvec = jnp.zeros((16,), jnp.int32)
            acc = [jnp.zeros((16,), jnp.float32) for _ in range(3)]
            for _ in range(_D):
                b = plsc.load_gather(in_v, [splat_e, dvec])
                for j in range(3):
                    v = plsc.load_gather(comb_v, [rows[j], dvec])
                    acc[j] = acc[j] + v * b
                dvec = dvec + 1
            for j in range(3):
                out_v[e, pl.ds(16 * j, 16)] = acc[j]
            return c2

        lax.fori_loop(0, _E, elem, 0)
        return pltpu.async_copy(out_v, out.at[pl.ds(wbase + g * 16, 16)], osem)

    pending = fire(0)
    out_cps = [None, None]
    for g in range(_G):
        nxt = fire(g + 1) if g + 1 < _G else []
        for cp in pending:
            cp.wait()
        pending = nxt
        if out_cps[g % 2] is not None:
            out_cps[g % 2].wait()
        out_cps[g % 2] = compute(g)
    for cp in out_cps:
        cp.wait()


def _tc_body(dots_ref, vs_ref, va_ref, out_ref):
    x = dots_ref[...]
    s1 = (jnp.sum(jax.nn.log_sigmoid(x[:, 0:10] ** 2))
          + jnp.sum(jax.nn.log_sigmoid(-(x[:, 10:30] ** 2))))
    syn = jnp.sum(x[:, 30:35], axis=1, keepdims=True) * vs_ref[...]
    ant = jnp.sum(x[:, 35:40], axis=1, keepdims=True) * va_ref[...]
    s2 = jnp.sum(ant - syn)
    out_ref[...] = jnp.concatenate(
        [jnp.reshape(s1, (1, 1)), jnp.reshape(s2, (1, 1))], axis=1)


_tc_reduce = pl.pallas_call(
    _tc_body,
    out_shape=jax.ShapeDtypeStruct((1, 2), jnp.float32),
)


def kernel(table, input_labels, pos_labels, neg_labels, syn_word_idxs,
           ant_word_idxs, valid_syn, valid_ant, upsilon, eta0, eta):
    table = table.astype(jnp.float32)
    ii = input_labels.astype(jnp.int32)
    dots = _sc_dots(ii,
                    pos_labels.astype(jnp.int32).reshape(-1),
                    neg_labels.astype(jnp.int32).reshape(-1),
                    syn_word_idxs.astype(jnp.int32).reshape(-1),
                    ant_word_idxs.astype(jnp.int32).reshape(-1),
                    table)
    s = _tc_reduce(dots,
                   valid_syn.astype(jnp.float32).reshape(_B, 1),
                   valid_ant.astype(jnp.float32).reshape(_B, 1))
    bf = jnp.float32(_B)
    loss = eta0 * (s[0, 0] / bf) - eta * jnp.maximum(
        jnp.float32(0.0), upsilon + s[0, 1] / bf)
    return -loss


# speedup vs baseline: 2.4838x; 1.0905x over previous
import functools

import jax
import jax.numpy as jnp
from jax import lax
from jax.experimental import pallas as pl
from jax.experimental.pallas import tpu as pltpu
from jax.experimental.pallas import tpu_sc as plsc

_D = 64
_B = 4096
_NW = 32
_PER_W = _B // _NW
_E = 16
_G = _PER_W // _E
_CROWS = 640
_OUTC = 40

_mesh = plsc.VectorSubcoreMesh(core_axis_name="c", subcore_axis_name="s")


@functools.partial(
    pl.kernel,
    mesh=_mesh,
    out_type=jax.ShapeDtypeStruct((_B * _OUTC,), jnp.float32),
    scratch_types=[
        pltpu.VMEM((_PER_W,), jnp.int32),
        pltpu.VMEM((_PER_W * 10,), jnp.int32),
        pltpu.VMEM((_PER_W * 20,), jnp.int32),
        pltpu.VMEM((_PER_W * 5,), jnp.int32),
        pltpu.VMEM((_PER_W * 5,), jnp.int32),
        pltpu.VMEM((_E, _D), jnp.float32),
        pltpu.VMEM((_E, _D), jnp.float32),
        pltpu.VMEM((_CROWS, _D), jnp.float32),
        pltpu.VMEM((_CROWS, _D), jnp.float32),
        pltpu.VMEM((_E * _OUTC,), jnp.float32),
        pltpu.VMEM((_E * _OUTC,), jnp.float32),
        pltpu.SemaphoreType.DMA,
        pltpu.SemaphoreType.DMA,
        pltpu.SemaphoreType.DMA,
    ],
    compiler_params=pltpu.CompilerParams(
        needs_layout_passes=False, use_tc_tiling_on_sc=False),
)
def _sc_dots(in_idx, pos_idx, neg_idx, syn_idx, ant_idx, table, out,
             iidx, pidx, nidx, sidx, aidx,
             in_v0, in_v1, cv0, cv1, ov0, ov1, sem0, sem1, osem):
    wid = lax.axis_index("s") * 2 + lax.axis_index("c")
    iota = lax.iota(jnp.int32, 16)

    m2 = []
    bb = []
    for q in range(5):
        cp = 16 * q + iota
        sel = jnp.where(cp < 40, 0, 1)
        c = cp - 40 * sel
        mult = jnp.where(c < 10, 10, jnp.where(c < 30, 20, 5))
        base = jnp.where(
            c < 10, c,
            jnp.where(c < 30, 150 + c, jnp.where(c < 35, 450 + c, 525 + c)))
        m2.append(2 * mult)
        bb.append(sel * mult + base)
    lane_lo = iota < 8
    il_const = [jnp.full((16, 1), l, jnp.int32) for l in range(16)]
    gd = lax.GatherDimensionNumbers(
        offset_dims=(), collapsed_slice_dims=(0,), start_index_map=(0,))

    def bcast_lane(x, il2):
        return lax.gather(x, il2, gd, (1,),
                          mode=lax.GatherScatterMode.PROMISE_IN_BOUNDS)

    wbase = wid * _PER_W
    pltpu.sync_copy(in_idx.at[pl.ds(wbase, _PER_W)], iidx)
    pltpu.sync_copy(pos_idx.at[pl.ds(wbase * 10, _PER_W * 10)], pidx)
    pltpu.sync_copy(neg_idx.at[pl.ds(wbase * 20, _PER_W * 20)], nidx)
    pltpu.sync_copy(syn_idx.at[pl.ds(wbase * 5, _PER_W * 5)], sidx)
    pltpu.sync_copy(ant_idx.at[pl.ds(wbase * 5, _PER_W * 5)], aidx)

    in_bufs = (in_v0, in_v1)
    c_bufs = (cv0, cv1)
    o_bufs = (ov0, ov1)
    sems = (sem0, sem1)

    def fire(g):
        cb, sem = c_bufs[g % 2], sems[g % 2]
        cps = [
            pltpu.async_copy(
                table.at[iidx.at[pl.ds(g * 16, 16)]], in_bufs[g % 2], sem),
            pltpu.async_copy(
                table.at[pidx.at[pl.ds(g * 160, 128)]],
                cb.at[pl.ds(0, 128)], sem),
            pltpu.async_copy(
                table.at[pidx.at[pl.ds(g * 160 + 128, 32)]],
                cb.at[pl.ds(128, 32)], sem),
            pltpu.async_copy(
                table.at[nidx.at[pl.ds(g * 320, 128)]],
                cb.at[pl.ds(160, 128)], sem),
            pltpu.async_copy(
                table.at[nidx.at[pl.ds(g * 320 + 128, 128)]],
                cb.at[pl.ds(288, 128)], sem),
            pltpu.async_copy(
                table.at[nidx.at[pl.ds(g * 320 + 256, 64)]],
                cb.at[pl.ds(416, 64)], sem),
            pltpu.async_copy(
                table.at[sidx.at[pl.ds(g * 80, 80)]],
                cb.at[pl.ds(480, 80)], sem),
            pltpu.async_copy(
                table.at[aidx.at[pl.ds(g * 80, 80)]],
                cb.at[pl.ds(560, 80)], sem),
        ]
        return cps

    def compute(g):
        in_v, comb_v, out_v = in_bufs[g % 2], c_bufs[g % 2], o_bufs[g % 2]

        def pair(t, c2):
            eA = t * 2
            rA = [in_v[eA, pl.ds(16 * j, 16)] for j in range(4)]
            rB = [in_v[eA + 1, pl.ds(16 * j, 16)] for j in range(4)]
            splat_t = jnp.full((16,), t, jnp.int32)
            rows = [splat_t * m2[q] + bb[q] for q in range(5)]
            dvec = jnp.zeros((16,), jnp.int32)
            acc = [jnp.zeros((16,), jnp.float32) for _ in range(5)]
            for d in range(_D):
                j, l = d // 16, d % 16
                bA = bcast_lane(rA[j], il_const[l])
                bB = bcast_lane(rB[j], il_const[l])
                bM = jnp.where(lane_lo, bA, bB)
                bs = (bA, bA, bM, bB, bB)
                for q in range(5):
                    v = plsc.load_gather(comb_v, [rows[q], dvec])
                    acc[q] = acc[q] + v * bs[q]
                dvec = dvec + 1
            for q in range(5):
                out_v[pl.ds(80 * t + 16 * q, 16)] = acc[q]
            return c2

        lax.fori_loop(0, _E // 2, pair, 0)
        return pltpu.async_copy(
            out_v, out.at[pl.ds((wbase + g * 16) * _OUTC, _E * _OUTC)], osem)

    pending = fire(0)
    out_cps = [None, None]
    for g in range(_G):
        nxt = fire(g + 1) if g + 1 < _G else []
        for cp in pending:
            cp.wait()
        pending = nxt
        if out_cps[g % 2] is not None:
            out_cps[g % 2].wait()
        out_cps[g % 2] = compute(g)
    for cp in out_cps:
        cp.wait()


def _tc_body(dots_ref, vs_ref, va_ref, out_ref):
    x = dots_ref[...]
    s1 = (jnp.sum(jax.nn.log_sigmoid(x[:, 0:10] ** 2))
          + jnp.sum(jax.nn.log_sigmoid(-(x[:, 10:30] ** 2))))
    syn = jnp.sum(x[:, 30:35], axis=1, keepdims=True) * vs_ref[...]
    ant = jnp.sum(x[:, 35:40], axis=1, keepdims=True) * va_ref[...]
    s2 = jnp.sum(ant - syn)
    out_ref[...] = jnp.concatenate(
        [jnp.reshape(s1, (1, 1)), jnp.reshape(s2, (1, 1))], axis=1)


_tc_reduce = pl.pallas_call(
    _tc_body,
    out_shape=jax.ShapeDtypeStruct((1, 2), jnp.float32),
)


def kernel(table, input_labels, pos_labels, neg_labels, syn_word_idxs,
           ant_word_idxs, valid_syn, valid_ant, upsilon, eta0, eta):
    table = table.astype(jnp.float32)
    ii = input_labels.astype(jnp.int32)
    dots = _sc_dots(ii,
                    pos_labels.astype(jnp.int32).reshape(-1),
                    neg_labels.astype(jnp.int32).reshape(-1),
                    syn_word_idxs.astype(jnp.int32).reshape(-1),
                    ant_word_idxs.astype(jnp.int32).reshape(-1),
                    table).reshape(_B, _OUTC)
    s = _tc_reduce(dots,
                   valid_syn.astype(jnp.float32).reshape(_B, 1),
                   valid_ant.astype(jnp.float32).reshape(_B, 1))
    bf = jnp.float32(_B)
    loss = eta0 * (s[0, 0] / bf) - eta * jnp.maximum(
        jnp.float32(0.0), upsilon + s[0, 1] / bf)
    return -loss


# speedup vs baseline: 2.5479x; 1.0258x over previous
import functools

import jax
import jax.numpy as jnp
from jax import lax
from jax.experimental import pallas as pl
from jax.experimental.pallas import tpu as pltpu
from jax.experimental.pallas import tpu_sc as plsc

_D = 64
_B = 4096
_NW = 32
_PER_W = _B // _NW
_E = 16
_G = _PER_W // _E
_CROWS = 640
_OUTC = 40

_mesh = plsc.VectorSubcoreMesh(core_axis_name="c", subcore_axis_name="s")


@functools.partial(
    pl.kernel,
    mesh=_mesh,
    out_type=jax.ShapeDtypeStruct((_B, _OUTC), jnp.float32),
    scratch_types=[
        pltpu.VMEM((_PER_W,), jnp.int32),
        pltpu.VMEM((_PER_W * 10,), jnp.int32),
        pltpu.VMEM((_PER_W * 20,), jnp.int32),
        pltpu.VMEM((_PER_W * 5,), jnp.int32),
        pltpu.VMEM((_PER_W * 5,), jnp.int32),
        pltpu.VMEM((_E, _D), jnp.float32),
        pltpu.VMEM((_E, _D), jnp.float32),
        pltpu.VMEM((_CROWS, _D), jnp.float32),
        pltpu.VMEM((_CROWS, _D), jnp.float32),
        pltpu.VMEM((_E, _OUTC), jnp.float32),
        pltpu.VMEM((_E, _OUTC), jnp.float32),
        pltpu.SemaphoreType.DMA,
        pltpu.SemaphoreType.DMA,
        pltpu.SemaphoreType.DMA,
    ],
    compiler_params=pltpu.CompilerParams(
        needs_layout_passes=False, use_tc_tiling_on_sc=False),
)
def _sc_dots(in_idx, pos_idx, neg_idx, syn_idx, ant_idx, table, out,
             iidx, pidx, nidx, sidx, aidx,
             in_v0, in_v1, cv0, cv1, ov0, ov1, sem0, sem1, osem):
    wid = lax.axis_index("s") * 2 + lax.axis_index("c")
    iota = lax.iota(jnp.int32, 16)

    m2 = []
    bb = []
    for q in range(5):
        cp = 16 * q + iota
        sel = jnp.where(cp < 40, 0, 1)
        c = cp - 40 * sel
        mult = jnp.where(c < 10, 10, jnp.where(c < 30, 20, 5))
        base = jnp.where(
            c < 10, c,
            jnp.where(c < 30, 150 + c, jnp.where(c < 35, 450 + c, 525 + c)))
        m2.append(2 * mult)
        bb.append(sel * mult + base)
    lane_lo = iota < 8
    il_const = [jnp.full((16, 1), l, jnp.int32) for l in range(16)]
    gd = lax.GatherDimensionNumbers(
        offset_dims=(), collapsed_slice_dims=(0,), start_index_map=(0,))

    def bcast_lane(x, il2):
        return lax.gather(x, il2, gd, (1,),
                          mode=lax.GatherScatterMode.PROMISE_IN_BOUNDS)

    q2_rowoff = jnp.where(lane_lo, 0, 1)
    q2_col = jnp.where(lane_lo, 32 + iota, iota - 8)

    wbase = wid * _PER_W
    pltpu.sync_copy(in_idx.at[pl.ds(wbase, _PER_W)], iidx)
    pltpu.sync_copy(pos_idx.at[pl.ds(wbase * 10, _PER_W * 10)], pidx)
    pltpu.sync_copy(neg_idx.at[pl.ds(wbase * 20, _PER_W * 20)], nidx)
    pltpu.sync_copy(syn_idx.at[pl.ds(wbase * 5, _PER_W * 5)], sidx)
    pltpu.sync_copy(ant_idx.at[pl.ds(wbase * 5, _PER_W * 5)], aidx)

    in_bufs = (in_v0, in_v1)
    c_bufs = (cv0, cv1)
    o_bufs = (ov0, ov1)
    sems = (sem0, sem1)

    def fire(g):
        cb, sem = c_bufs[g % 2], sems[g % 2]
        cps = [
            pltpu.async_copy(
                table.at[iidx.at[pl.ds(g * 16, 16)]], in_bufs[g % 2], sem),
            pltpu.async_copy(
                table.at[pidx.at[pl.ds(g * 160, 128)]],
                cb.at[pl.ds(0, 128)], sem),
            pltpu.async_copy(
                table.at[pidx.at[pl.ds(g * 160 + 128, 32)]],
                cb.at[pl.ds(128, 32)], sem),
            pltpu.async_copy(
                table.at[nidx.at[pl.ds(g * 320, 128)]],
                cb.at[pl.ds(160, 128)], sem),
            pltpu.async_copy(
                table.at[nidx.at[pl.ds(g * 320 + 128, 128)]],
                cb.at[pl.ds(288, 128)], sem),
            pltpu.async_copy(
                table.at[nidx.at[pl.ds(g * 320 + 256, 64)]],
                cb.at[pl.ds(416, 64)], sem),
            pltpu.async_copy(
                table.at[sidx.at[pl.ds(g * 80, 80)]],
                cb.at[pl.ds(480, 80)], sem),
            pltpu.async_copy(
                table.at[aidx.at[pl.ds(g * 80, 80)]],
                cb.at[pl.ds(560, 80)], sem),
        ]
        return cps

    def compute(g):
        in_v, comb_v, out_v = in_bufs[g % 2], c_bufs[g % 2], o_bufs[g % 2]

        def pair(t, c2):
            eA = t * 2
            rA = [in_v[eA, pl.ds(16 * j, 16)] for j in range(4)]
            rB = [in_v[eA + 1, pl.ds(16 * j, 16)] for j in range(4)]
            splat_t = jnp.full((16,), t, jnp.int32)
            rows = [splat_t * m2[q] + bb[q] for q in range(5)]
            dvec = jnp.zeros((16,), jnp.int32)
            acc = [jnp.zeros((16,), jnp.float32) for _ in range(5)]
            for d in range(_D):
                j, l = d // 16, d % 16
                bA = bcast_lane(rA[j], il_const[l])
                bB = bcast_lane(rB[j], il_const[l])
                bM = jnp.where(lane_lo, bA, bB)
                bs = (bA, bA, bM, bB, bB)
                for q in range(5):
                    v = plsc.load_gather(comb_v, [rows[q], dvec])
                    acc[q] = acc[q] + v * bs[q]
                dvec = dvec + 1
            out_v[eA, pl.ds(0, 16)] = acc[0]
            out_v[eA, pl.ds(16, 16)] = acc[1]
            plsc.store_scatter(
                out_v, [jnp.full((16,), eA, jnp.int32) + q2_rowoff, q2_col],
                acc[2])
            out_v[eA + 1, pl.ds(8, 16)] = acc[3]
            out_v[eA + 1, pl.ds(24, 16)] = acc[4]
            return c2

        lax.fori_loop(0, _E // 2, pair, 0)
        return pltpu.async_copy(
            out_v, out.at[pl.ds(wbase + g * 16, _E)], osem)

    pending = fire(0)
    out_cps = [None, None]
    for g in range(_G):
        nxt = fire(g + 1) if g + 1 < _G else []
        for cp in pending:
            cp.wait()
        pending = nxt
        if out_cps[g % 2] is not None:
            out_cps[g % 2].wait()
        out_cps[g % 2] = compute(g)
    for cp in out_cps:
        cp.wait()


def _tc_body(dots_ref, vs_ref, va_ref, out_ref):
    x = dots_ref[...]
    s1 = (jnp.sum(jax.nn.log_sigmoid(x[:, 0:10] ** 2))
          + jnp.sum(jax.nn.log_sigmoid(-(x[:, 10:30] ** 2))))
    syn = jnp.sum(x[:, 30:35], axis=1, keepdims=True) * vs_ref[...]
    ant = jnp.sum(x[:, 35:40], axis=1, keepdims=True) * va_ref[...]
    s2 = jnp.sum(ant - syn)
    out_ref[...] = jnp.concatenate(
        [jnp.reshape(s1, (1, 1)), jnp.reshape(s2, (1, 1))], axis=1)


_tc_reduce = pl.pallas_call(
    _tc_body,
    out_shape=jax.ShapeDtypeStruct((1, 2), jnp.float32),
)


def kernel(table, input_labels, pos_labels, neg_labels, syn_word_idxs,
           ant_word_idxs, valid_syn, valid_ant, upsilon, eta0, eta):
    table = table.astype(jnp.float32)
    ii = input_labels.astype(jnp.int32)
    dots = _sc_dots(ii,
                    pos_labels.astype(jnp.int32).reshape(-1),
                    neg_labels.astype(jnp.int32).reshape(-1),
                    syn_word_idxs.astype(jnp.int32).reshape(-1),
                    ant_word_idxs.astype(jnp.int32).reshape(-1),
                    table)
    s = _tc_reduce(dots,
                   valid_syn.astype(jnp.float32).reshape(_B, 1),
                   valid_ant.astype(jnp.float32).reshape(_B, 1))
    bf = jnp.float32(_B)
    loss = eta0 * (s[0, 0] / bf) - eta * jnp.maximum(
        jnp.float32(0.0), upsilon + s[0, 1] / bf)
    return -loss


# speedup vs baseline: 2.6046x; 1.0223x over previous
import functools

import jax
import jax.numpy as jnp
from jax import lax
from jax.experimental import pallas as pl
from jax.experimental.pallas import tpu as pltpu
from jax.experimental.pallas import tpu_sc as plsc

_D = 64
_B = 4096
_NW = 32
_PER_W = _B // _NW
_E = 16
_G = _PER_W // _E
_CROWS = 640
_OUTC = 40

_mesh = plsc.VectorSubcoreMesh(core_axis_name="c", subcore_axis_name="s")


@functools.partial(
    pl.kernel,
    mesh=_mesh,
    out_type=(jax.ShapeDtypeStruct((_B * _OUTC,), jnp.float32),
              jax.ShapeDtypeStruct((_NW, 16), jnp.float32)),
    scratch_types=[
        pltpu.VMEM((_PER_W,), jnp.int32),
        pltpu.VMEM((_PER_W * 10,), jnp.int32),
        pltpu.VMEM((_PER_W * 20,), jnp.int32),
        pltpu.VMEM((_PER_W * 5,), jnp.int32),
        pltpu.VMEM((_PER_W * 5,), jnp.int32),
        pltpu.VMEM((_E, _D), jnp.float32),
        pltpu.VMEM((_E, _D), jnp.float32),
        pltpu.VMEM((_CROWS, _D), jnp.float32),
        pltpu.VMEM((_CROWS, _D), jnp.float32),
        pltpu.VMEM((_E * _OUTC,), jnp.float32),
        pltpu.VMEM((_E * _OUTC,), jnp.float32),
        pltpu.VMEM((_PER_W,), jnp.float32),
        pltpu.VMEM((_PER_W,), jnp.float32),
        pltpu.VMEM((16,), jnp.float32),
        pltpu.SemaphoreType.DMA,
        pltpu.SemaphoreType.DMA,
        pltpu.SemaphoreType.DMA,
    ],
    compiler_params=pltpu.CompilerParams(
        needs_layout_passes=False, use_tc_tiling_on_sc=False),
)
def _sc_dots(in_idx, pos_idx, neg_idx, syn_idx, ant_idx, vsyn, vant, table,
             out, out2, iidx, pidx, nidx, sidx, aidx,
             in_v0, in_v1, cv0, cv1, ov0, ov1, vs_v, va_v, s2v,
             sem0, sem1, osem):
    wid = lax.axis_index("s") * 2 + lax.axis_index("c")
    iota = lax.iota(jnp.int32, 16)

    m2 = []
    bb = []
    for q in range(5):
        cp = 16 * q + iota
        sel = jnp.where(cp < 40, 0, 1)
        c = cp - 40 * sel
        mult = jnp.where(c < 10, 10, jnp.where(c < 30, 20, 5))
        base = jnp.where(
            c < 10, c,
            jnp.where(c < 30, 150 + c, jnp.where(c < 35, 450 + c, 525 + c)))
        m2.append(2 * mult)
        bb.append(sel * mult + base)
    lane_lo = iota < 8
    il_const = [jnp.full((16, 1), l, jnp.int32) for l in range(16)]
    gd = lax.GatherDimensionNumbers(
        offset_dims=(), collapsed_slice_dims=(0,), start_index_map=(0,))

    def bcast_lane(x, il2):
        return lax.gather(x, il2, gd, (1,),
                          mode=lax.GatherScatterMode.PROMISE_IN_BOUNDS)

    fz = jnp.float32(0.0)
    m1s = jnp.where(iota >= 14, jnp.float32(-1.0), fz)
    m2s = jnp.where(iota < 3, jnp.float32(-1.0), fz)
    m2a = jnp.where((iota >= 3) & (iota < 8), jnp.float32(1.0), fz)
    m4s = jnp.where((iota >= 6) & (iota < 11), jnp.float32(-1.0), fz)
    m4a = jnp.where(iota >= 11, jnp.float32(1.0), fz)

    wbase = wid * _PER_W
    pltpu.sync_copy(in_idx.at[pl.ds(wbase, _PER_W)], iidx)
    pltpu.sync_copy(pos_idx.at[pl.ds(wbase * 10, _PER_W * 10)], pidx)
    pltpu.sync_copy(neg_idx.at[pl.ds(wbase * 20, _PER_W * 20)], nidx)
    pltpu.sync_copy(syn_idx.at[pl.ds(wbase * 5, _PER_W * 5)], sidx)
    pltpu.sync_copy(ant_idx.at[pl.ds(wbase * 5, _PER_W * 5)], aidx)
    pltpu.sync_copy(vsyn.at[pl.ds(wbase, _PER_W)], vs_v)
    pltpu.sync_copy(vant.at[pl.ds(wbase, _PER_W)], va_v)
    s2v[...] = jnp.zeros((16,), jnp.float32)

    in_bufs = (in_v0, in_v1)
    c_bufs = (cv0, cv1)
    o_bufs = (ov0, ov1)
    sems = (sem0, sem1)

    def fire(g):
        cb, sem = c_bufs[g % 2], sems[g % 2]
        cps = [
            pltpu.async_copy(
                table.at[iidx.at[pl.ds(g * 16, 16)]], in_bufs[g % 2], sem),
            pltpu.async_copy(
                table.at[pidx.at[pl.ds(g * 160, 128)]],
                cb.at[pl.ds(0, 128)], sem),
            pltpu.async_copy(
                table.at[pidx.at[pl.ds(g * 160 + 128, 32)]],
                cb.at[pl.ds(128, 32)], sem),
            pltpu.async_copy(
                table.at[nidx.at[pl.ds(g * 320, 128)]],
                cb.at[pl.ds(160, 128)], sem),
            pltpu.async_copy(
                table.at[nidx.at[pl.ds(g * 320 + 128, 128)]],
                cb.at[pl.ds(288, 128)], sem),
            pltpu.async_copy(
                table.at[nidx.at[pl.ds(g * 320 + 256, 64)]],
                cb.at[pl.ds(416, 64)], sem),
            pltpu.async_copy(
                table.at[sidx.at[pl.ds(g * 80, 80)]],
                cb.at[pl.ds(480, 80)], sem),
            pltpu.async_copy(
                table.at[aidx.at[pl.ds(g * 80, 80)]],
                cb.at[pl.ds(560, 80)], sem),
        ]
        return cps

    def compute(g):
        in_v, comb_v, out_v = in_bufs[g % 2], c_bufs[g % 2], o_bufs[g % 2]

        def pair(t, c2):
            eA = t * 2
            rA = [in_v[eA, pl.ds(16 * j, 16)] for j in range(4)]
            rB = [in_v[eA + 1, pl.ds(16 * j, 16)] for j in range(4)]
            splat_t = jnp.full((16,), t, jnp.int32)
            rows = [splat_t * m2[q] + bb[q] for q in range(5)]
            dvec = jnp.zeros((16,), jnp.int32)
            acc = [jnp.zeros((16,), jnp.float32) for _ in range(5)]
            for d in range(_D):
                j, l = d // 16, d % 16
                bA = bcast_lane(rA[j], il_const[l])
                bB = bcast_lane(rB[j], il_const[l])
                bM = jnp.where(lane_lo, bA, bB)
                bs = (bA, bA, bM, bB, bB)
                for q in range(5):
                    v = plsc.load_gather(comb_v, [rows[q], dvec])
                    acc[q] = acc[q] + v * bs[q]
                dvec = dvec + 1
            for q in range(5):
                out_v[pl.ds(80 * t + 16 * q, 16)] = acc[q]
            we = jnp.full((16,), g * 16, jnp.int32) + eA
            vsA = plsc.load_gather(vs_v, [we])
            vaA = plsc.load_gather(va_v, [we])
            vsB = plsc.load_gather(vs_v, [we + 1])
            vaB = plsc.load_gather(va_v, [we + 1])
            s2v[...] = (s2v[...] + acc[1] * (vsA * m1s)
                        + acc[2] * (vsA * m2s + vaA * m2a)
                        + acc[4] * (vsB * m4s + vaB * m4a))
            return c2

        lax.fori_loop(0, _E // 2, pair, 0)
        return pltpu.async_copy(
            out_v, out.at[pl.ds((wbase + g * 16) * _OUTC, _E * _OUTC)], osem)

    pending = fire(0)
    out_cps = [None, None]
    for g in range(_G):
        nxt = fire(g + 1) if g + 1 < _G else []
        for cp in pending:
            cp.wait()
        pending = nxt
        if out_cps[g % 2] is not None:
            out_cps[g % 2].wait()
        out_cps[g % 2] = compute(g)
    for cp in out_cps:
        cp.wait()
    pltpu.sync_copy(s2v, out2.at[wid])


_TCR = _B * _OUTC // 128


def _tc_body(dots_ref, s2_ref, out_ref):
    x = dots_ref[...]
    i0 = lax.broadcasted_iota(jnp.int32, (_TCR, 128), 0)
    i1 = lax.broadcasted_iota(jnp.int32, (_TCR, 128), 1)
    c = (i0 * 128 + i1) % _OUTC
    xsq = x * x
    val = jax.nn.log_sigmoid(jnp.where(c < 10, xsq, -xsq))
    s1 = jnp.sum(jnp.where(c < 30, val, jnp.float32(0.0)))
    s2 = jnp.sum(s2_ref[...])
    out_ref[...] = jnp.concatenate(
        [jnp.reshape(s1, (1, 1)), jnp.reshape(s2, (1, 1))], axis=1)


_tc_reduce = pl.pallas_call(
    _tc_body,
    out_shape=jax.ShapeDtypeStruct((1, 2), jnp.float32),
)


def kernel(table, input_labels, pos_labels, neg_labels, syn_word_idxs,
           ant_word_idxs, valid_syn, valid_ant, upsilon, eta0, eta):
    table = table.astype(jnp.float32)
    ii = input_labels.astype(jnp.int32)
    dots, s2p = _sc_dots(ii,
                         pos_labels.astype(jnp.int32).reshape(-1),
                         neg_labels.astype(jnp.int32).reshape(-1),
                         syn_word_idxs.astype(jnp.int32).reshape(-1),
                         ant_word_idxs.astype(jnp.int32).reshape(-1),
                         valid_syn.astype(jnp.float32),
                         valid_ant.astype(jnp.float32),
                         table)
    s = _tc_reduce(dots.reshape(_TCR, 128), s2p)
    bf = jnp.float32(_B)
    loss = eta0 * (s[0, 0] / bf) - eta * jnp.maximum(
        jnp.float32(0.0), upsilon + s[0, 1] / bf)
    return -loss


# speedup vs baseline: 4.3260x; 1.6609x over previous
import functools

import jax
import jax.numpy as jnp
from jax import lax
from jax.experimental import pallas as pl
from jax.experimental.pallas import tpu as pltpu
from jax.experimental.pallas import tpu_sc as plsc

_D = 64
_B = 4096
_NW = 32
_PER_W = _B // _NW
_E = 16
_G = _PER_W // _E
_CROWS = 640
_OUTC = 40

_mesh = plsc.VectorSubcoreMesh(core_axis_name="c", subcore_axis_name="s")


@functools.partial(
    pl.kernel,
    mesh=_mesh,
    out_type=(jax.ShapeDtypeStruct((_B * _OUTC,), jnp.float32),
              jax.ShapeDtypeStruct((_NW, 16), jnp.float32)),
    scratch_types=[
        pltpu.VMEM((_PER_W,), jnp.int32),
        pltpu.VMEM((_PER_W * 10,), jnp.int32),
        pltpu.VMEM((_PER_W * 20,), jnp.int32),
        pltpu.VMEM((_PER_W * 5,), jnp.int32),
        pltpu.VMEM((_PER_W * 5,), jnp.int32),
        pltpu.VMEM((_E, _D), jnp.float32),
        pltpu.VMEM((_E, _D), jnp.float32),
        pltpu.VMEM((_CROWS, _D), jnp.float32),
        pltpu.VMEM((_CROWS, _D), jnp.float32),
        pltpu.VMEM((_E * _OUTC,), jnp.float32),
        pltpu.VMEM((_E * _OUTC,), jnp.float32),
        pltpu.VMEM((_PER_W,), jnp.float32),
        pltpu.VMEM((_PER_W,), jnp.float32),
        pltpu.VMEM((16,), jnp.float32),
        pltpu.VMEM((16 * 17,), jnp.float32),
        pltpu.SemaphoreType.DMA,
        pltpu.SemaphoreType.DMA,
        pltpu.SemaphoreType.DMA,
    ],
    compiler_params=pltpu.CompilerParams(
        needs_layout_passes=False, use_tc_tiling_on_sc=False),
)
def _sc_dots(in_idx, pos_idx, neg_idx, syn_idx, ant_idx, vsyn, vant, table,
             out, out2, iidx, pidx, nidx, sidx, aidx,
             in_v0, in_v1, cv0, cv1, ov0, ov1, vs_v, va_v, s2v, ptile,
             sem0, sem1, osem):
    wid = lax.axis_index("s") * 2 + lax.axis_index("c")
    iota = lax.iota(jnp.int32, 16)

    def row_mb(cp):
        soff, c = cp // 40, cp % 40
        if c < 10:
            m, b = 10, c
        elif c < 30:
            m, b = 20, 150 + c
        elif c < 35:
            m, b = 5, 450 + c
        else:
            m, b = 5, 525 + c
        return 2 * m, soff * m + b, soff

    tcol = [17 * iota + c for c in range(16)]

    fz = jnp.float32(0.0)
    m1s = jnp.where(iota >= 14, jnp.float32(-1.0), fz)
    m2s = jnp.where(iota < 3, jnp.float32(-1.0), fz)
    m2a = jnp.where((iota >= 3) & (iota < 8), jnp.float32(1.0), fz)
    m4s = jnp.where((iota >= 6) & (iota < 11), jnp.float32(-1.0), fz)
    m4a = jnp.where(iota >= 11, jnp.float32(1.0), fz)

    wbase = wid * _PER_W
    pltpu.sync_copy(in_idx.at[pl.ds(wbase, _PER_W)], iidx)
    pltpu.sync_copy(pos_idx.at[pl.ds(wbase * 10, _PER_W * 10)], pidx)
    pltpu.sync_copy(neg_idx.at[pl.ds(wbase * 20, _PER_W * 20)], nidx)
    pltpu.sync_copy(syn_idx.at[pl.ds(wbase * 5, _PER_W * 5)], sidx)
    pltpu.sync_copy(ant_idx.at[pl.ds(wbase * 5, _PER_W * 5)], aidx)
    pltpu.sync_copy(vsyn.at[pl.ds(wbase, _PER_W)], vs_v)
    pltpu.sync_copy(vant.at[pl.ds(wbase, _PER_W)], va_v)
    s2v[...] = jnp.zeros((16,), jnp.float32)

    in_bufs = (in_v0, in_v1)
    c_bufs = (cv0, cv1)
    o_bufs = (ov0, ov1)
    sems = (sem0, sem1)

    def fire(g):
        cb, sem = c_bufs[g % 2], sems[g % 2]
        cps = [
            pltpu.async_copy(
                table.at[iidx.at[pl.ds(g * 16, 16)]], in_bufs[g % 2], sem),
            pltpu.async_copy(
                table.at[pidx.at[pl.ds(g * 160, 128)]],
                cb.at[pl.ds(0, 128)], sem),
            pltpu.async_copy(
                table.at[pidx.at[pl.ds(g * 160 + 128, 32)]],
                cb.at[pl.ds(128, 32)], sem),
            pltpu.async_copy(
                table.at[nidx.at[pl.ds(g * 320, 128)]],
                cb.at[pl.ds(160, 128)], sem),
            pltpu.async_copy(
                table.at[nidx.at[pl.ds(g * 320 + 128, 128)]],
                cb.at[pl.ds(288, 128)], sem),
            pltpu.async_copy(
                table.at[nidx.at[pl.ds(g * 320 + 256, 64)]],
                cb.at[pl.ds(416, 64)], sem),
            pltpu.async_copy(
                table.at[sidx.at[pl.ds(g * 80, 80)]],
                cb.at[pl.ds(480, 80)], sem),
            pltpu.async_copy(
                table.at[aidx.at[pl.ds(g * 80, 80)]],
                cb.at[pl.ds(560, 80)], sem),
        ]
        return cps

    def compute(g):
        in_v, comb_v, out_v = in_bufs[g % 2], c_bufs[g % 2], o_bufs[g % 2]

        def pair(t, c2):
            eA = t * 2
            ivs = [[in_v[eA + s, pl.ds(16 * j, 16)] for j in range(4)]
                   for s in range(2)]
            acc = []
            for q in range(5):
                for k in range(16):
                    m2c, bc, soff = row_mb(16 * q + k)
                    row = t * m2c + bc
                    iv = ivs[soff]
                    cv = [comb_v[row, pl.ds(16 * j, 16)] for j in range(4)]
                    prod = (cv[0] * iv[0] + cv[1] * iv[1]
                            + cv[2] * iv[2] + cv[3] * iv[3])
                    ptile[pl.ds(17 * k, 16)] = prod
                tot = plsc.load_gather(ptile, [tcol[0]])
                for c in range(1, 16):
                    tot = tot + plsc.load_gather(ptile, [tcol[c]])
                out_v[pl.ds(80 * t + 16 * q, 16)] = tot
                acc.append(tot)
            we = jnp.full((16,), g * 16, jnp.int32) + eA
            vsA = plsc.load_gather(vs_v, [we])
            vaA = plsc.load_gather(va_v, [we])
            vsB = plsc.load_gather(vs_v, [we + 1])
            vaB = plsc.load_gather(va_v, [we + 1])
            s2v[...] = (s2v[...] + acc[1] * (vsA * m1s)
                        + acc[2] * (vsA * m2s + vaA * m2a)
                        + acc[4] * (vsB * m4s + vaB * m4a))
            return c2

        lax.fori_loop(0, _E // 2, pair, 0)
        return pltpu.async_copy(
            out_v, out.at[pl.ds((wbase + g * 16) * _OUTC, _E * _OUTC)], osem)

    pending = fire(0)
    out_cps = [None, None]
    for g in range(_G):
        nxt = fire(g + 1) if g + 1 < _G else []
        for cp in pending:
            cp.wait()
        pending = nxt
        if out_cps[g % 2] is not None:
            out_cps[g % 2].wait()
        out_cps[g % 2] = compute(g)
    for cp in out_cps:
        cp.wait()
    pltpu.sync_copy(s2v, out2.at[wid])


_TCR = _B * _OUTC // 128


def _tc_body(dots_ref, s2_ref, out_ref):
    x = dots_ref[...]
    i0 = lax.broadcasted_iota(jnp.int32, (_TCR, 128), 0)
    i1 = lax.broadcasted_iota(jnp.int32, (_TCR, 128), 1)
    c = (i0 * 128 + i1) % _OUTC
    xsq = x * x
    val = jax.nn.log_sigmoid(jnp.where(c < 10, xsq, -xsq))
    s1 = jnp.sum(jnp.where(c < 30, val, jnp.float32(0.0)))
    s2 = jnp.sum(s2_ref[...])
    out_ref[...] = jnp.concatenate(
        [jnp.reshape(s1, (1, 1)), jnp.reshape(s2, (1, 1))], axis=1)


_tc_reduce = pl.pallas_call(
    _tc_body,
    out_shape=jax.ShapeDtypeStruct((1, 2), jnp.float32),
)


def kernel(table, input_labels, pos_labels, neg_labels, syn_word_idxs,
           ant_word_idxs, valid_syn, valid_ant, upsilon, eta0, eta):
    table = table.astype(jnp.float32)
    ii = input_labels.astype(jnp.int32)
    dots, s2p = _sc_dots(ii,
                         pos_labels.astype(jnp.int32).reshape(-1),
                         neg_labels.astype(jnp.int32).reshape(-1),
                         syn_word_idxs.astype(jnp.int32).reshape(-1),
                         ant_word_idxs.astype(jnp.int32).reshape(-1),
                         valid_syn.astype(jnp.float32),
                         valid_ant.astype(jnp.float32),
                         table)
    s = _tc_reduce(dots.reshape(_TCR, 128), s2p)
    bf = jnp.float32(_B)
    loss = eta0 * (s[0, 0] / bf) - eta * jnp.maximum(
        jnp.float32(0.0), upsilon + s[0, 1] / bf)
    return -loss


# speedup vs baseline: 4.3908x; 1.0150x over previous
import functools

import jax
import jax.numpy as jnp
from jax import lax
from jax.experimental import pallas as pl
from jax.experimental.pallas import tpu as pltpu
from jax.experimental.pallas import tpu_sc as plsc

_D = 64
_B = 4096
_NW = 32
_PER_W = _B // _NW
_E = 16
_G = _PER_W // _E
_CROWS = 640
_OUTC = 40

_mesh = plsc.VectorSubcoreMesh(core_axis_name="c", subcore_axis_name="s")


@functools.partial(
    pl.kernel,
    mesh=_mesh,
    out_type=(jax.ShapeDtypeStruct((_B * _OUTC,), jnp.float32),
              jax.ShapeDtypeStruct((_NW, 16), jnp.float32)),
    scratch_types=[
        pltpu.VMEM((_PER_W,), jnp.int32),
        pltpu.VMEM((_PER_W * 10,), jnp.int32),
        pltpu.VMEM((_PER_W * 20,), jnp.int32),
        pltpu.VMEM((_PER_W * 5,), jnp.int32),
        pltpu.VMEM((_PER_W * 5,), jnp.int32),
        pltpu.VMEM((_E, _D), jnp.float32),
        pltpu.VMEM((_E, _D), jnp.float32),
        pltpu.VMEM((_CROWS, _D), jnp.float32),
        pltpu.VMEM((_CROWS, _D), jnp.float32),
        pltpu.VMEM((_E * _OUTC,), jnp.float32),
        pltpu.VMEM((_E * _OUTC,), jnp.float32),
        pltpu.VMEM((_PER_W,), jnp.float32),
        pltpu.VMEM((_PER_W,), jnp.float32),
        pltpu.VMEM((16,), jnp.float32),
        pltpu.VMEM((16 * 17,), jnp.float32),
        pltpu.VMEM((16 * 17,), jnp.float32),
        pltpu.SemaphoreType.DMA,
        pltpu.SemaphoreType.DMA,
        pltpu.SemaphoreType.DMA,
    ],
    compiler_params=pltpu.CompilerParams(
        needs_layout_passes=False, use_tc_tiling_on_sc=False),
)
def _sc_dots(in_idx, pos_idx, neg_idx, syn_idx, ant_idx, vsyn, vant, table,
             out, out2, iidx, pidx, nidx, sidx, aidx,
             in_v0, in_v1, cv0, cv1, ov0, ov1, vs_v, va_v, s2v, pt0, pt1,
             sem0, sem1, osem):
    wid = lax.axis_index("s") * 2 + lax.axis_index("c")
    iota = lax.iota(jnp.int32, 16)

    def row_mb(cp):
        soff, c = cp // 40, cp % 40
        if c < 10:
            m, b = 10, c
        elif c < 30:
            m, b = 20, 150 + c
        elif c < 35:
            m, b = 5, 450 + c
        else:
            m, b = 5, 525 + c
        return 2 * m, soff * m + b, soff

    tcol = [17 * iota + c for c in range(16)]

    fz = jnp.float32(0.0)
    m1s = jnp.where(iota >= 14, jnp.float32(-1.0), fz)
    m2s = jnp.where(iota < 3, jnp.float32(-1.0), fz)
    m2a = jnp.where((iota >= 3) & (iota < 8), jnp.float32(1.0), fz)
    m4s = jnp.where((iota >= 6) & (iota < 11), jnp.float32(-1.0), fz)
    m4a = jnp.where(iota >= 11, jnp.float32(1.0), fz)

    wbase = wid * _PER_W
    pltpu.sync_copy(in_idx.at[pl.ds(wbase, _PER_W)], iidx)
    pltpu.sync_copy(pos_idx.at[pl.ds(wbase * 10, _PER_W * 10)], pidx)
    pltpu.sync_copy(neg_idx.at[pl.ds(wbase * 20, _PER_W * 20)], nidx)
    pltpu.sync_copy(syn_idx.at[pl.ds(wbase * 5, _PER_W * 5)], sidx)
    pltpu.sync_copy(ant_idx.at[pl.ds(wbase * 5, _PER_W * 5)], aidx)
    pltpu.sync_copy(vsyn.at[pl.ds(wbase, _PER_W)], vs_v)
    pltpu.sync_copy(vant.at[pl.ds(wbase, _PER_W)], va_v)
    s2v[...] = jnp.zeros((16,), jnp.float32)

    in_bufs = (in_v0, in_v1)
    c_bufs = (cv0, cv1)
    o_bufs = (ov0, ov1)
    sems = (sem0, sem1)

    def fire(g):
        cb, sem = c_bufs[g % 2], sems[g % 2]
        cps = [
            pltpu.async_copy(
                table.at[iidx.at[pl.ds(g * 16, 16)]], in_bufs[g % 2], sem),
            pltpu.async_copy(
                table.at[pidx.at[pl.ds(g * 160, 128)]],
                cb.at[pl.ds(0, 128)], sem),
            pltpu.async_copy(
                table.at[pidx.at[pl.ds(g * 160 + 128, 32)]],
                cb.at[pl.ds(128, 32)], sem),
            pltpu.async_copy(
                table.at[nidx.at[pl.ds(g * 320, 128)]],
                cb.at[pl.ds(160, 128)], sem),
            pltpu.async_copy(
                table.at[nidx.at[pl.ds(g * 320 + 128, 128)]],
                cb.at[pl.ds(288, 128)], sem),
            pltpu.async_copy(
                table.at[nidx.at[pl.ds(g * 320 + 256, 64)]],
                cb.at[pl.ds(416, 64)], sem),
            pltpu.async_copy(
                table.at[sidx.at[pl.ds(g * 80, 80)]],
                cb.at[pl.ds(480, 80)], sem),
            pltpu.async_copy(
                table.at[aidx.at[pl.ds(g * 80, 80)]],
                cb.at[pl.ds(560, 80)], sem),
        ]
        return cps

    def compute(g):
        in_v, comb_v, out_v = in_bufs[g % 2], c_bufs[g % 2], o_bufs[g % 2]

        def one_pair(t, ptile):
            eA = t * 2
            ivs = [[in_v[eA + s, pl.ds(16 * j, 16)] for j in range(4)]
                   for s in range(2)]
            acc = []
            for q in range(5):
                for k in range(16):
                    m2c, bc, soff = row_mb(16 * q + k)
                    row = t * m2c + bc
                    iv = ivs[soff]
                    cv = [comb_v[row, pl.ds(16 * j, 16)] for j in range(4)]
                    prod = ((cv[0] * iv[0] + cv[1] * iv[1])
                            + (cv[2] * iv[2] + cv[3] * iv[3]))
                    ptile[pl.ds(17 * k, 16)] = prod
                gs = [plsc.load_gather(ptile, [tcol[c]]) for c in range(16)]
                while len(gs) > 1:
                    gs = [gs[i] + gs[i + 1] for i in range(0, len(gs), 2)]
                tot = gs[0]
                out_v[pl.ds(80 * t + 16 * q, 16)] = tot
                acc.append(tot)
            we = jnp.full((16,), g * 16, jnp.int32) + eA
            vsA = plsc.load_gather(vs_v, [we])
            vaA = plsc.load_gather(va_v, [we])
            vsB = plsc.load_gather(vs_v, [we + 1])
            vaB = plsc.load_gather(va_v, [we + 1])
            return (acc[1] * (vsA * m1s)
                    + acc[2] * (vsA * m2s + vaA * m2a)
                    + acc[4] * (vsB * m4s + vaB * m4a))

        def pair(t, c2):
            s2v[...] = s2v[...] + one_pair(t, pt0)
            return c2

        lax.fori_loop(0, _E // 2, pair, 0)
        return pltpu.async_copy(
            out_v, out.at[pl.ds((wbase + g * 16) * _OUTC, _E * _OUTC)], osem)

    pending = fire(0)
    out_cps = [None, None]
    for g in range(_G):
        nxt = fire(g + 1) if g + 1 < _G else []
        for cp in pending:
            cp.wait()
        pending = nxt
        if out_cps[g % 2] is not None:
            out_cps[g % 2].wait()
        out_cps[g % 2] = compute(g)
    for cp in out_cps:
        cp.wait()
    pltpu.sync_copy(s2v, out2.at[wid])


_TCR = _B * _OUTC // 128


def _tc_body(dots_ref, s2_ref, out_ref):
    x = dots_ref[...]
    i0 = lax.broadcasted_iota(jnp.int32, (_TCR, 128), 0)
    i1 = lax.broadcasted_iota(jnp.int32, (_TCR, 128), 1)
    c = (i0 * 128 + i1) % _OUTC
    xsq = x * x
    val = jax.nn.log_sigmoid(jnp.where(c < 10, xsq, -xsq))
    s1 = jnp.sum(jnp.where(c < 30, val, jnp.float32(0.0)))
    s2 = jnp.sum(s2_ref[...])
    out_ref[...] = jnp.concatenate(
        [jnp.reshape(s1, (1, 1)), jnp.reshape(s2, (1, 1))], axis=1)


_tc_reduce = pl.pallas_call(
    _tc_body,
    out_shape=jax.ShapeDtypeStruct((1, 2), jnp.float32),
)


def kernel(table, input_labels, pos_labels, neg_labels, syn_word_idxs,
           ant_word_idxs, valid_syn, valid_ant, upsilon, eta0, eta):
    table = table.astype(jnp.float32)
    ii = input_labels.astype(jnp.int32)
    dots, s2p = _sc_dots(ii,
                         pos_labels.astype(jnp.int32).reshape(-1),
                         neg_labels.astype(jnp.int32).reshape(-1),
                         syn_word_idxs.astype(jnp.int32).reshape(-1),
                         ant_word_idxs.astype(jnp.int32).reshape(-1),
                         valid_syn.astype(jnp.float32),
                         valid_ant.astype(jnp.float32),
                         table)
    s = _tc_reduce(dots.reshape(_TCR, 128), s2p)
    bf = jnp.float32(_B)
    loss = eta0 * (s[0, 0] / bf) - eta * jnp.maximum(
        jnp.float32(0.0), upsilon + s[0, 1] / bf)
    return -loss


# speedup vs baseline: 4.3921x; 1.0003x over previous
import functools

import jax
import jax.numpy as jnp
from jax import lax
from jax.experimental import pallas as pl
from jax.experimental.pallas import tpu as pltpu
from jax.experimental.pallas import tpu_sc as plsc

_D = 64
_B = 4096
_NW = 32
_PER_W = _B // _NW
_E = 16
_G = _PER_W // _E
_CROWS = 640
_OUTC = 40

_mesh = plsc.VectorSubcoreMesh(core_axis_name="c", subcore_axis_name="s")


@functools.partial(
    pl.kernel,
    mesh=_mesh,
    out_type=(jax.ShapeDtypeStruct((_B * _OUTC,), jnp.float32),
              jax.ShapeDtypeStruct((_NW, 16), jnp.float32)),
    scratch_types=[
        pltpu.VMEM((_PER_W,), jnp.int32),
        pltpu.VMEM((_PER_W * 10,), jnp.int32),
        pltpu.VMEM((_PER_W * 20,), jnp.int32),
        pltpu.VMEM((_PER_W * 5,), jnp.int32),
        pltpu.VMEM((_PER_W * 5,), jnp.int32),
        pltpu.VMEM((_E, _D), jnp.float32),
        pltpu.VMEM((_E, _D), jnp.float32),
        pltpu.VMEM((_CROWS, _D), jnp.float32),
        pltpu.VMEM((_CROWS, _D), jnp.float32),
        pltpu.VMEM((_E * _OUTC,), jnp.float32),
        pltpu.VMEM((_E * _OUTC,), jnp.float32),
        pltpu.VMEM((_PER_W,), jnp.float32),
        pltpu.VMEM((_PER_W,), jnp.float32),
        pltpu.VMEM((16,), jnp.float32),
        pltpu.VMEM((16 * 17,), jnp.float32),
        pltpu.SemaphoreType.DMA,
        pltpu.SemaphoreType.DMA,
        pltpu.SemaphoreType.DMA,
    ],
    compiler_params=pltpu.CompilerParams(
        needs_layout_passes=False, use_tc_tiling_on_sc=False),
)
def _sc_dots(in_idx, pos_idx, neg_idx, syn_idx, ant_idx, vsyn, vant, table,
             out, out2, iidx, pidx, nidx, sidx, aidx,
             in_v0, in_v1, cv0, cv1, ov0, ov1, vs_v, va_v, s2v, pt0,
             sem0, sem1, osem):
    wid = lax.axis_index("s") * 2 + lax.axis_index("c")
    iota = lax.iota(jnp.int32, 16)

    def row_mb(cp):
        soff, c = cp // 40, cp % 40
        if c < 10:
            m, b = 10, c
        elif c < 30:
            m, b = 20, 150 + c
        elif c < 35:
            m, b = 5, 450 + c
        else:
            m, b = 5, 525 + c
        return 2 * m, soff * m + b, soff

    tcol = [17 * iota + c for c in range(16)]

    fz = jnp.float32(0.0)
    m1s = jnp.where(iota >= 14, jnp.float32(-1.0), fz)
    m2s = jnp.where(iota < 3, jnp.float32(-1.0), fz)
    m2a = jnp.where((iota >= 3) & (iota < 8), jnp.float32(1.0), fz)
    m4s = jnp.where((iota >= 6) & (iota < 11), jnp.float32(-1.0), fz)
    m4a = jnp.where(iota >= 11, jnp.float32(1.0), fz)

    wbase = wid * _PER_W
    pltpu.sync_copy(in_idx.at[pl.ds(wbase, _PER_W)], iidx)
    pltpu.sync_copy(pos_idx.at[pl.ds(wbase * 10, _PER_W * 10)], pidx)
    pltpu.sync_copy(neg_idx.at[pl.ds(wbase * 20, _PER_W * 20)], nidx)
    pltpu.sync_copy(syn_idx.at[pl.ds(wbase * 5, _PER_W * 5)], sidx)
    pltpu.sync_copy(ant_idx.at[pl.ds(wbase * 5, _PER_W * 5)], aidx)
    pltpu.sync_copy(vsyn.at[pl.ds(wbase, _PER_W)], vs_v)
    pltpu.sync_copy(vant.at[pl.ds(wbase, _PER_W)], va_v)
    s2v[...] = jnp.zeros((16,), jnp.float32)

    in_bufs = (in_v0, in_v1)
    c_bufs = (cv0, cv1)
    o_bufs = (ov0, ov1)
    sems = (sem0, sem1)

    def fire(g):
        cb, sem = c_bufs[g % 2], sems[g % 2]
        cps = [
            pltpu.async_copy(
                table.at[iidx.at[pl.ds(g * 16, 16)]], in_bufs[g % 2], sem),
            pltpu.async_copy(
                table.at[pidx.at[pl.ds(g * 160, 128)]],
                cb.at[pl.ds(0, 128)], sem),
            pltpu.async_copy(
                table.at[pidx.at[pl.ds(g * 160 + 128, 32)]],
                cb.at[pl.ds(128, 32)], sem),
            pltpu.async_copy(
                table.at[nidx.at[pl.ds(g * 320, 128)]],
                cb.at[pl.ds(160, 128)], sem),
            pltpu.async_copy(
                table.at[nidx.at[pl.ds(g * 320 + 128, 128)]],
                cb.at[pl.ds(288, 128)], sem),
            pltpu.async_copy(
                table.at[nidx.at[pl.ds(g * 320 + 256, 64)]],
                cb.at[pl.ds(416, 64)], sem),
            pltpu.async_copy(
                table.at[sidx.at[pl.ds(g * 80, 80)]],
                cb.at[pl.ds(480, 80)], sem),
            pltpu.async_copy(
                table.at[aidx.at[pl.ds(g * 80, 80)]],
                cb.at[pl.ds(560, 80)], sem),
        ]
        return cps

    def compute(g):
        in_v, comb_v, out_v = in_bufs[g % 2], c_bufs[g % 2], o_bufs[g % 2]

        def one_pair(t, ptile):
            eA = t * 2
            ivs = [[in_v[eA + s, pl.ds(16 * j, 16)] for j in range(4)]
                   for s in range(2)]
            acc = []
            for q in range(5):
                for k in range(16):
                    m2c, bc, soff = row_mb(16 * q + k)
                    row = t * m2c + bc
                    iv = ivs[soff]
                    cv = [comb_v[row, pl.ds(16 * j, 16)] for j in range(4)]
                    prod = ((cv[0] * iv[0] + cv[1] * iv[1])
                            + (cv[2] * iv[2] + cv[3] * iv[3]))
                    ptile[pl.ds(17 * k, 16)] = prod
                gs = [plsc.load_gather(ptile, [tcol[c]]) for c in range(16)]
                while len(gs) > 1:
                    gs = [gs[i] + gs[i + 1] for i in range(0, len(gs), 2)]
                tot = gs[0]
                out_v[pl.ds(80 * t + 16 * q, 16)] = tot
                acc.append(tot)
            we = jnp.full((16,), g * 16, jnp.int32) + eA
            vsA = plsc.load_gather(vs_v, [we])
            vaA = plsc.load_gather(va_v, [we])
            vsB = plsc.load_gather(vs_v, [we + 1])
            vaB = plsc.load_gather(va_v, [we + 1])
            return (acc[1] * (vsA * m1s)
                    + acc[2] * (vsA * m2s + vaA * m2a)
                    + acc[4] * (vsB * m4s + vaB * m4a))

        def pair(t, c2):
            s2v[...] = s2v[...] + one_pair(t, pt0)
            return c2

        lax.fori_loop(0, _E // 2, pair, 0)
        return pltpu.async_copy(
            out_v, out.at[pl.ds((wbase + g * 16) * _OUTC, _E * _OUTC)], osem)

    pending = fire(0)
    out_cps = [None, None]
    for g in range(_G):
        nxt = fire(g + 1) if g + 1 < _G else []
        for cp in pending:
            cp.wait()
        pending = nxt
        if out_cps[g % 2] is not None:
            out_cps[g % 2].wait()
        out_cps[g % 2] = compute(g)
    for cp in out_cps:
        cp.wait()
    pltpu.sync_copy(s2v, out2.at[wid])


_TCR = _B * _OUTC // 128


def _tc_body(dots_ref, s2_ref, out_ref):
    x = dots_ref[...]
    i0 = lax.broadcasted_iota(jnp.int32, (_TCR, 128), 0)
    i1 = lax.broadcasted_iota(jnp.int32, (_TCR, 128), 1)
    c = (i0 * 128 + i1) % _OUTC
    xsq = x * x
    val = jax.nn.log_sigmoid(jnp.where(c < 10, xsq, -xsq))
    s1 = jnp.sum(jnp.where(c < 30, val, jnp.float32(0.0)))
    s2 = jnp.sum(s2_ref[...])
    out_ref[...] = jnp.concatenate(
        [jnp.reshape(s1, (1, 1)), jnp.reshape(s2, (1, 1))], axis=1)


_tc_reduce = pl.pallas_call(
    _tc_body,
    out_shape=jax.ShapeDtypeStruct((1, 2), jnp.float32),
)


def kernel(table, input_labels, pos_labels, neg_labels, syn_word_idxs,
           ant_word_idxs, valid_syn, valid_ant, upsilon, eta0, eta):
    table = table.astype(jnp.float32)
    ii = input_labels.astype(jnp.int32)
    dots, s2p = _sc_dots(ii,
                         pos_labels.astype(jnp.int32).reshape(-1),
                         neg_labels.astype(jnp.int32).reshape(-1),
                         syn_word_idxs.astype(jnp.int32).reshape(-1),
                         ant_word_idxs.astype(jnp.int32).reshape(-1),
                         valid_syn.astype(jnp.float32),
                         valid_ant.astype(jnp.float32),
                         table)
    s = _tc_reduce(dots.reshape(_TCR, 128), s2p)
    bf = jnp.float32(_B)
    loss = eta0 * (s[0, 0] / bf) - eta * jnp.maximum(
        jnp.float32(0.0), upsilon + s[0, 1] / bf)
    return -loss
